# baseline (speedup 1.0000x reference)
import jax
import jax.numpy as jnp
from jax import lax
from jax.experimental import pallas as pl
from jax.experimental.pallas import tpu as pltpu
from jax.experimental.pallas import tpu_sc as plsc

_T = 8192
_ROWS = 4096
_NC = 2
_NS = 16
_NW = _NC * _NS
_TR = 8
_TL = 128
_NQ = 8
_QTOT = _T // _TL
_PROWS = _ROWS // _TR
_PPW = _PROWS // _NW
_CPW = _PPW * (_QTOT // _NQ)
_CW = _NQ * _TL


def _fix(inbuf, outbuf):
    iota = lax.iota(jnp.int32, 16)
    idx8 = iota + 8
    rows = [jnp.full((16,), r, jnp.int32) for r in range(_TR)]
    cross = [iota + (z * _TL + 120) for z in range(_NQ)]
    def gather_group(z, r):
        base = z * _TL
        vals = []
        for k in range(8):
            if k == 6:
                vals.append(plsc.load_gather(inbuf, [rows[r], cross[z]]))
            else:
                window = inbuf.at[r, pl.ds(base + 16 * (k + 1), 32)]
                vals.append(plsc.load_gather(window, [idx8]))
        return vals

    def store_group(z, r, vals):
        base = z * _TL
        for k in range(8):
            outbuf[r, pl.ds(base + 16 * k, 16)] = vals[k]

    groups = [(z, r) for z in range(_NQ) for r in range(0, _TR, 2)]
    prev = None
    for z, r in groups:
        vals = gather_group(z, r) + gather_group(z, r + 1)
        if prev is not None:
            pz, pr, pv = prev
            store_group(pz, pr, pv[:8])
            store_group(pz, pr + 1, pv[8:])
        prev = (z, r, vals)
    pz, pr, pv = prev
    store_group(pz, pr, pv[:8])
    store_group(pz, pr + 1, pv[8:])


def _roll_body(x_hbm, out_hbm, in0, in1, out0, out1,
               la0, la1, lb0, lb1, so0, so1):
    wid = lax.axis_index("s") * _NC + lax.axis_index("c")
    p0 = wid * _PPW
    inbufs = (in0, in1)
    outbufs = (out0, out1)
    lsems_a = (la0, la1)
    lsems_b = (lb0, lb1)
    ssems = (so0, so1)

    def addrs(i):
        p = p0 + i // (_QTOT // _NQ)
        m = i % (_QTOT // _NQ)
        rows = p * _TR
        col_out = m * _CW
        col_src = ((m + _QTOT // _NQ - 1) % (_QTOT // _NQ)) * _CW
        return rows, col_out, col_src

    def load(i, s):
        rows, col_out, col_src = addrs(i)
        return (
            pltpu.make_async_copy(
                x_hbm.at[pl.ds(rows, _TR), pl.ds(col_src, _CW)],
                inbufs[s].at[:, pl.ds(0, _CW)],
                lsems_a[s],
            ),
            pltpu.make_async_copy(
                x_hbm.at[pl.ds(rows, _TR), pl.ds(col_out, _TL)],
                inbufs[s].at[:, pl.ds(_CW, _TL)],
                lsems_b[s],
            ),
        )

    def store(i, s):
        rows, col_out, _ = addrs(i)
        return pltpu.make_async_copy(
            outbufs[s],
            out_hbm.at[pl.ds(rows, _TR), pl.ds(col_out, _CW)],
            ssems[s],
        )

    def start_load(i, s):
        a, b = load(i, s)
        a.start()
        b.start()

    def step(i, s, wait_store, prefetch):
        a, b = load(i, s)
        a.wait()
        b.wait()
        if wait_store:
            store(i - 2, s).wait()
        _fix(inbufs[s], outbufs[s])
        store(i, s).start()
        if prefetch:
            start_load(i + 2, s)

    start_load(0, 0)
    start_load(1, 1)
    step(0, 0, False, True)
    step(1, 1, False, True)

    def group(g, carry):
        step(2 * g, 0, True, True)
        step(2 * g + 1, 1, True, True)
        return carry

    lax.fori_loop(1, _CPW // 2 - 1, group, 0)

    step(_CPW - 2, 0, True, False)
    step(_CPW - 1, 1, True, False)
    store(_CPW - 2, 0).wait()
    store(_CPW - 1, 1).wait()


def kernel(x):
    b, s, t = x.shape
    x2 = x.reshape(b * s, t)
    mesh = plsc.VectorSubcoreMesh(core_axis_name="c", subcore_axis_name="s")
    out = pl.kernel(
        _roll_body,
        out_type=jax.ShapeDtypeStruct((b * s, t), x.dtype),
        mesh=mesh,
        scratch_types=(
            pltpu.VMEM((_TR, _CW + _TL), jnp.float32),
            pltpu.VMEM((_TR, _CW + _TL), jnp.float32),
            pltpu.VMEM((_TR, _CW), jnp.float32),
            pltpu.VMEM((_TR, _CW), jnp.float32),
            pltpu.SemaphoreType.DMA,
            pltpu.SemaphoreType.DMA,
            pltpu.SemaphoreType.DMA,
            pltpu.SemaphoreType.DMA,
            pltpu.SemaphoreType.DMA,
            pltpu.SemaphoreType.DMA,
        ),
        compiler_params=pltpu.CompilerParams(needs_layout_passes=False),
    )(x2)
    return out.reshape(b, s, t)


# speedup vs baseline: 1.0410x; 1.0410x over previous
import jax
import jax.numpy as jnp
from jax import lax
from jax.experimental import pallas as pl
from jax.experimental.pallas import tpu as pltpu
from jax.experimental.pallas import tpu_sc as plsc

_T = 8192
_ROWS = 4096
_NC = 2
_NS = 16
_NW = _NC * _NS
_TR = 8
_TL = 128
_NQ = 8
_QTOT = _T // _TL
_PROWS = _ROWS // _TR
_PPW = _PROWS // _NW
_CPW = _PPW * (_QTOT // _NQ)
_CW = _NQ * _TL


def _fix(inbuf, outbuf):
    iota = lax.iota(jnp.int32, 16)
    idx8 = iota + 8
    rows = [jnp.full((16,), r, jnp.int32) for r in range(_TR)]
    cross = [iota + (z * _TL + 120) for z in range(_NQ)]
    def gather_group(z, r):
        base = z * _TL
        vals = []
        for k in range(8):
            if k == 6:
                vals.append(plsc.load_gather(inbuf, [rows[r], cross[z]]))
            else:
                window = inbuf.at[r, pl.ds(base + 16 * (k + 1), 32)]
                vals.append(plsc.load_gather(window, [idx8]))
        return vals

    def store_group(z, r, vals):
        base = z * _TL
        for k in range(8):
            outbuf[r, pl.ds(base + 16 * k, 16)] = vals[k]

    for z in range(_NQ):
        for r in range(_TR):
            store_group(z, r, gather_group(z, r))


def _roll_body(x_hbm, out_hbm, in0, in1, out0, out1,
               la0, la1, lb0, lb1, so0, so1):
    wid = lax.axis_index("s") * _NC + lax.axis_index("c")
    p0 = wid * _PPW
    inbufs = (in0, in1)
    outbufs = (out0, out1)
    lsems_a = (la0, la1)
    lsems_b = (lb0, lb1)
    ssems = (so0, so1)

    def addrs(i):
        p = p0 + i // (_QTOT // _NQ)
        m = i % (_QTOT // _NQ)
        rows = p * _TR
        col_out = m * _CW
        col_src = ((m + _QTOT // _NQ - 1) % (_QTOT // _NQ)) * _CW
        return rows, col_out, col_src

    def load(i, s):
        rows, col_out, col_src = addrs(i)
        return (
            pltpu.make_async_copy(
                x_hbm.at[pl.ds(rows, _TR), pl.ds(col_src, _CW)],
                inbufs[s].at[:, pl.ds(0, _CW)],
                lsems_a[s],
            ),
            pltpu.make_async_copy(
                x_hbm.at[pl.ds(rows, _TR), pl.ds(col_out, _TL)],
                inbufs[s].at[:, pl.ds(_CW, _TL)],
                lsems_b[s],
            ),
        )

    def store(i, s):
        rows, col_out, _ = addrs(i)
        return pltpu.make_async_copy(
            outbufs[s],
            out_hbm.at[pl.ds(rows, _TR), pl.ds(col_out, _CW)],
            ssems[s],
        )

    def start_load(i, s):
        a, b = load(i, s)
        a.start()
        b.start()

    def step(i, s, wait_store, prefetch):
        a, b = load(i, s)
        a.wait()
        b.wait()
        if wait_store:
            store(i - 2, s).wait()
        _fix(inbufs[s], outbufs[s])
        store(i, s).start()
        if prefetch:
            start_load(i + 2, s)

    start_load(0, 0)
    start_load(1, 1)
    step(0, 0, False, True)
    step(1, 1, False, True)

    def group(g, carry):
        step(2 * g, 0, True, True)
        step(2 * g + 1, 1, True, True)
        return carry

    lax.fori_loop(1, _CPW // 2 - 1, group, 0)

    step(_CPW - 2, 0, True, False)
    step(_CPW - 1, 1, True, False)
    store(_CPW - 2, 0).wait()
    store(_CPW - 1, 1).wait()


def kernel(x):
    b, s, t = x.shape
    x2 = x.reshape(b * s, t)
    mesh = plsc.VectorSubcoreMesh(core_axis_name="c", subcore_axis_name="s")
    out = pl.kernel(
        _roll_body,
        out_type=jax.ShapeDtypeStruct((b * s, t), x.dtype),
        mesh=mesh,
        scratch_types=(
            pltpu.VMEM((_TR, _CW + _TL), jnp.float32),
            pltpu.VMEM((_TR, _CW + _TL), jnp.float32),
            pltpu.VMEM((_TR, _CW), jnp.float32),
            pltpu.VMEM((_TR, _CW), jnp.float32),
            pltpu.SemaphoreType.DMA,
            pltpu.SemaphoreType.DMA,
            pltpu.SemaphoreType.DMA,
            pltpu.SemaphoreType.DMA,
            pltpu.SemaphoreType.DMA,
            pltpu.SemaphoreType.DMA,
        ),
        compiler_params=pltpu.CompilerParams(needs_layout_passes=False),
    )(x2)
    return out.reshape(b, s, t)
